# baseline (speedup 1.0000x reference)
import functools

import jax
import jax.numpy as jnp
from jax import lax
from jax.experimental import pallas as pl
from jax.experimental.pallas import tpu as pltpu
from jax.experimental.pallas import tpu_sc as plsc

_NUM_SERIES = 100000
_SEASONALITY = 24
_BATCH = 4096
_PACK = 8


@functools.lru_cache(maxsize=None)
def _build_gather():
    info = plsc.get_sparse_core_info()
    nc, ns, nl = info.num_cores, info.num_subcores, info.num_lanes
    nw = nc * ns
    b_per_w = _BATCH // nw
    n_grp = b_per_w // nl
    mesh = plsc.VectorSubcoreMesh(core_axis_name="c", subcore_axis_name="s")

    @functools.partial(
        pl.kernel,
        mesh=mesh,
        out_type=(
            jax.ShapeDtypeStruct((_BATCH,), jnp.float32),
            jax.ShapeDtypeStruct((_BATCH,), jnp.float32),
            jax.ShapeDtypeStruct((_BATCH, _SEASONALITY), jnp.float32),
        ),
        scratch_types=[
            pltpu.VMEM((b_per_w,), jnp.int32),
            pltpu.VMEM((b_per_w,), jnp.int32),
            pltpu.VMEM((b_per_w, _PACK), jnp.float32),
            pltpu.VMEM((b_per_w, _PACK), jnp.float32),
            pltpu.VMEM((b_per_w, _SEASONALITY), jnp.float32),
            pltpu.VMEM((b_per_w,), jnp.float32),
            pltpu.VMEM((b_per_w,), jnp.float32),
            pltpu.SemaphoreType.DMA,
            pltpu.SemaphoreType.DMA,
            pltpu.SemaphoreType.DMA,
        ],
        compiler_params=pltpu.CompilerParams(
            use_tc_tiling_on_sc=False, needs_layout_passes=False),
    )
    def gather_kernel(lev_hbm, seas_hbm, season_hbm, idx_hbm,
                      lev_out, seas_out, season_out,
                      idx_v, row_v, lev_rows, seas_rows, season_v,
                      lev_v, seas_v, sem0, sem1, sem2):
        wid = lax.axis_index("s") * nc + lax.axis_index("c")
        base = wid * b_per_w
        pltpu.sync_copy(idx_hbm.at[pl.ds(base, b_per_w)], idx_v)
        for j in range(n_grp):
            sl = pl.ds(j * nl, nl)
            row_v[sl] = lax.shift_right_logical(idx_v[sl], 3)
        c2 = pltpu.async_copy(season_hbm.at[idx_v], season_v, sem2)
        c0 = pltpu.async_copy(lev_hbm.at[row_v], lev_rows, sem0)
        c1 = pltpu.async_copy(seas_hbm.at[row_v], seas_rows, sem1)
        c0.wait()
        c1.wait()
        pos0 = lax.iota(jnp.int32, nl)
        for j in range(n_grp):
            sl = pl.ds(j * nl, nl)
            lane = lax.rem(idx_v[sl], _PACK)
            pos = pos0 + j * nl
            lev_v[sl] = plsc.load_gather(lev_rows, [pos, lane])
            seas_v[sl] = plsc.load_gather(seas_rows, [pos, lane])
        pltpu.sync_copy(lev_v, lev_out.at[pl.ds(base, b_per_w)])
        pltpu.sync_copy(seas_v, seas_out.at[pl.ds(base, b_per_w)])
        c2.wait()
        pltpu.sync_copy(season_v, season_out.at[pl.ds(base, b_per_w)])

    return gather_kernel


def kernel(train, val, test, info_cat, idxs, add_nl_layer,
           init_lev_sms, init_seas_sms, init_seasonalities):
    idx32 = idxs.astype(jnp.int32)
    gather = _build_gather()
    lev_flat, seas_flat, seasonalities = gather(
        init_lev_sms.reshape(_NUM_SERIES // _PACK, _PACK),
        init_seas_sms.reshape(_NUM_SERIES // _PACK, _PACK),
        init_seasonalities,
        idx32)
    return (lev_flat.reshape(_BATCH, 1), seas_flat.reshape(_BATCH, 1),
            seasonalities)


# speedup vs baseline: 1.3222x; 1.3222x over previous
import functools

import jax
import jax.numpy as jnp
from jax import lax
from jax.experimental import pallas as pl
from jax.experimental.pallas import tpu as pltpu
from jax.experimental.pallas import tpu_sc as plsc

N = 100000
S = 24
B = 4096
NC, NS = 2, 16
NW = NC * NS
BPW = B // NW
PACK = 8
RING = 64


@functools.lru_cache(maxsize=None)
def _build_season():
    mesh = plsc.VectorSubcoreMesh(core_axis_name="c", subcore_axis_name="s")

    @functools.partial(
        pl.kernel,
        mesh=mesh,
        out_type=jax.ShapeDtypeStruct((B, S), jnp.float32),
        scratch_types=[
            pltpu.VMEM((BPW,), jnp.int32),
            pltpu.VMEM((RING * PACK, S), jnp.float32),
            pltpu.VMEM((BPW, S), jnp.float32),
            pltpu.SemaphoreType.DMA,
        ],
        compiler_params=pltpu.CompilerParams(
            use_tc_tiling_on_sc=True, needs_layout_passes=False),
    )
    def season_kernel(tab_hbm, idx_hbm, out_hbm, idx_v, ring, srows, sem):
        wid = lax.axis_index("s") * NC + lax.axis_index("c")
        base = wid * BPW
        pltpu.sync_copy(idx_hbm.at[pl.ds(base, BPW)], idx_v)
        lanes = lax.iota(jnp.int32, 16)
        for rnd in range(BPW // RING):
            j0 = rnd * RING
            copies = []
            for g16 in range(RING // 16):
                vec = idx_v[pl.ds(j0 + g16 * 16, 16)]
                for l in range(16):
                    j = g16 * 16 + l
                    i = vec[l]
                    row8 = pl.multiple_of((i >> 3) << 3, 8)
                    copies.append(pltpu.async_copy(
                        tab_hbm.at[pl.ds(row8, PACK), :],
                        ring.at[pl.ds(j * PACK, PACK), :], sem))
            for c in copies:
                c.wait()
            for g in range(RING // 16):
                sl = pl.ds(j0 + g * 16, 16)
                sub = lax.rem(idx_v[sl], PACK)
                rowv = (g * 16 + lanes) * PACK + sub
                posv = j0 + g * 16 + lanes
                for c in range(S):
                    cv = jnp.full((16,), c, jnp.int32)
                    vals = plsc.load_gather(ring, [rowv, cv])
                    plsc.store_scatter(srows, [posv, cv], vals)
        pltpu.sync_copy(srows, out_hbm.at[pl.ds(base, BPW), :])

    return season_kernel


@functools.lru_cache(maxsize=None)
def _build_small():
    mesh = plsc.VectorSubcoreMesh(core_axis_name="c", subcore_axis_name="s")

    @functools.partial(
        pl.kernel,
        mesh=mesh,
        out_type=(
            jax.ShapeDtypeStruct((B,), jnp.float32),
            jax.ShapeDtypeStruct((B,), jnp.float32),
        ),
        scratch_types=[
            pltpu.VMEM((BPW,), jnp.int32),
            pltpu.VMEM((BPW,), jnp.int32),
            pltpu.VMEM((BPW, PACK), jnp.float32),
            pltpu.VMEM((BPW, PACK), jnp.float32),
            pltpu.VMEM((BPW,), jnp.float32),
            pltpu.VMEM((BPW,), jnp.float32),
            pltpu.SemaphoreType.DMA,
            pltpu.SemaphoreType.DMA,
        ],
        compiler_params=pltpu.CompilerParams(
            use_tc_tiling_on_sc=False, needs_layout_passes=False),
    )
    def small_kernel(lev_hbm, seas_hbm, idx_hbm, lev_out, seas_out,
                     idx_v, row_v, lev_rows, seas_rows, lev_v, seas_v,
                     sem0, sem1):
        wid = lax.axis_index("s") * NC + lax.axis_index("c")
        base = wid * BPW
        pltpu.sync_copy(idx_hbm.at[pl.ds(base, BPW)], idx_v)
        for j in range(BPW // 16):
            sl = pl.ds(j * 16, 16)
            row_v[sl] = lax.shift_right_logical(idx_v[sl], 3)
        c0 = pltpu.async_copy(lev_hbm.at[row_v], lev_rows, sem0)
        c1 = pltpu.async_copy(seas_hbm.at[row_v], seas_rows, sem1)
        c0.wait()
        c1.wait()
        pos0 = lax.iota(jnp.int32, 16)
        for j in range(BPW // 16):
            sl = pl.ds(j * 16, 16)
            lane = lax.rem(idx_v[sl], PACK)
            pos = pos0 + j * 16
            lev_v[sl] = plsc.load_gather(lev_rows, [pos, lane])
            seas_v[sl] = plsc.load_gather(seas_rows, [pos, lane])
        pltpu.sync_copy(lev_v, lev_out.at[pl.ds(base, BPW)])
        pltpu.sync_copy(seas_v, seas_out.at[pl.ds(base, BPW)])

    return small_kernel


def kernel(train, val, test, info_cat, idxs, add_nl_layer,
           init_lev_sms, init_seas_sms, init_seasonalities):
    idx32 = idxs.astype(jnp.int32)
    season = _build_season()(init_seasonalities, idx32)
    lev_flat, seas_flat = _build_small()(
        init_lev_sms.reshape(N // PACK, PACK),
        init_seas_sms.reshape(N // PACK, PACK), idx32)
    return (lev_flat.reshape(B, 1), seas_flat.reshape(B, 1), season)


# speedup vs baseline: 1.3652x; 1.0325x over previous
import functools

import jax
import jax.numpy as jnp
from jax import lax
from jax.experimental import pallas as pl
from jax.experimental.pallas import tpu as pltpu
from jax.experimental.pallas import tpu_sc as plsc

N = 100000
S = 24
B = 4096
NC, NS = 2, 16
NW = NC * NS
BPW = B // NW
PACK = 8
RING = 32


@functools.lru_cache(maxsize=None)
def _build_season():
    mesh = plsc.VectorSubcoreMesh(core_axis_name="c", subcore_axis_name="s")

    @functools.partial(
        pl.kernel,
        mesh=mesh,
        out_type=jax.ShapeDtypeStruct((B, S), jnp.float32),
        scratch_types=[
            pltpu.VMEM((BPW,), jnp.int32),
            pltpu.VMEM((2, RING * PACK, S), jnp.float32),
            pltpu.VMEM((BPW, S), jnp.float32),
            pltpu.SemaphoreType.DMA,
            pltpu.SemaphoreType.DMA,
        ],
        compiler_params=pltpu.CompilerParams(
            use_tc_tiling_on_sc=True, needs_layout_passes=False),
    )
    def season_kernel(tab_hbm, idx_hbm, out_hbm, idx_v, ring, srows,
                      sem0, sem1):
        wid = lax.axis_index("s") * NC + lax.axis_index("c")
        base = wid * BPW
        pltpu.sync_copy(idx_hbm.at[pl.ds(base, BPW)], idx_v)
        lanes = lax.iota(jnp.int32, 16)
        n_rounds = BPW // RING
        sems = [sem0, sem1]

        def fire(rnd):
            j0 = rnd * RING
            buf = rnd % 2
            copies = []
            for g16 in range(RING // 16):
                vec = idx_v[pl.ds(j0 + g16 * 16, 16)]
                for l in range(16):
                    j = g16 * 16 + l
                    i = vec[l]
                    row8 = pl.multiple_of((i >> 3) << 3, 8)
                    copies.append(pltpu.async_copy(
                        tab_hbm.at[pl.ds(row8, PACK), :],
                        ring.at[buf, pl.ds(j * PACK, PACK), :], sems[buf]))
            return copies

        def extract(rnd, copies):
            for c in copies:
                c.wait()
            j0 = rnd * RING
            buf = rnd % 2
            for g in range(RING // 16):
                sl = pl.ds(j0 + g * 16, 16)
                sub = lax.rem(idx_v[sl], PACK)
                rowv = (g * 16 + lanes) * PACK + sub
                bufv = jnp.full((16,), buf, jnp.int32)
                posv = j0 + g * 16 + lanes
                for c in range(S):
                    cv = jnp.full((16,), c, jnp.int32)
                    vals = plsc.load_gather(ring, [bufv, rowv, cv])
                    plsc.store_scatter(srows, [posv, cv], vals)

        pending = fire(0)
        for rnd in range(n_rounds):
            nxt = fire(rnd + 1) if rnd + 1 < n_rounds else None
            extract(rnd, pending)
            pending = nxt
        pltpu.sync_copy(srows, out_hbm.at[pl.ds(base, BPW), :])

    return season_kernel


@functools.lru_cache(maxsize=None)
def _build_small():
    mesh = plsc.VectorSubcoreMesh(core_axis_name="c", subcore_axis_name="s")

    @functools.partial(
        pl.kernel,
        mesh=mesh,
        out_type=(
            jax.ShapeDtypeStruct((B,), jnp.float32),
            jax.ShapeDtypeStruct((B,), jnp.float32),
        ),
        scratch_types=[
            pltpu.VMEM((BPW,), jnp.int32),
            pltpu.VMEM((BPW,), jnp.int32),
            pltpu.VMEM((BPW, PACK), jnp.float32),
            pltpu.VMEM((BPW, PACK), jnp.float32),
            pltpu.VMEM((BPW,), jnp.float32),
            pltpu.VMEM((BPW,), jnp.float32),
            pltpu.SemaphoreType.DMA,
            pltpu.SemaphoreType.DMA,
        ],
        compiler_params=pltpu.CompilerParams(
            use_tc_tiling_on_sc=False, needs_layout_passes=False),
    )
    def small_kernel(lev_hbm, seas_hbm, idx_hbm, lev_out, seas_out,
                     idx_v, row_v, lev_rows, seas_rows, lev_v, seas_v,
                     sem0, sem1):
        wid = lax.axis_index("s") * NC + lax.axis_index("c")
        base = wid * BPW
        pltpu.sync_copy(idx_hbm.at[pl.ds(base, BPW)], idx_v)
        for j in range(BPW // 16):
            sl = pl.ds(j * 16, 16)
            row_v[sl] = lax.shift_right_logical(idx_v[sl], 3)
        c0 = pltpu.async_copy(lev_hbm.at[row_v], lev_rows, sem0)
        c1 = pltpu.async_copy(seas_hbm.at[row_v], seas_rows, sem1)
        c0.wait()
        c1.wait()
        pos0 = lax.iota(jnp.int32, 16)
        for j in range(BPW // 16):
            sl = pl.ds(j * 16, 16)
            lane = lax.rem(idx_v[sl], PACK)
            pos = pos0 + j * 16
            lev_v[sl] = plsc.load_gather(lev_rows, [pos, lane])
            seas_v[sl] = plsc.load_gather(seas_rows, [pos, lane])
        pltpu.sync_copy(lev_v, lev_out.at[pl.ds(base, BPW)])
        pltpu.sync_copy(seas_v, seas_out.at[pl.ds(base, BPW)])

    return small_kernel


def kernel(train, val, test, info_cat, idxs, add_nl_layer,
           init_lev_sms, init_seas_sms, init_seasonalities):
    idx32 = idxs.astype(jnp.int32)
    lev_flat, seas_flat = _build_small()(
        init_lev_sms.reshape(N // PACK, PACK),
        init_seas_sms.reshape(N // PACK, PACK), idx32)
    season = _build_season()(init_seasonalities, idx32)
    return (lev_flat.reshape(B, 1), seas_flat.reshape(B, 1), season)


# speedup vs baseline: 1.4311x; 1.0483x over previous
import functools

import jax
import jax.numpy as jnp
from jax import lax
from jax.experimental import pallas as pl
from jax.experimental.pallas import tpu as pltpu
from jax.experimental.pallas import tpu_sc as plsc

N = 100000
S = 24
B = 4096
NC, NS = 2, 16
NW = NC * NS
BPW = B // NW
PACK = 8
RING = 32


@functools.lru_cache(maxsize=None)
def _build_season():
    mesh = plsc.VectorSubcoreMesh(core_axis_name="c", subcore_axis_name="s")

    @functools.partial(
        pl.kernel,
        mesh=mesh,
        out_type=jax.ShapeDtypeStruct((B, S), jnp.float32),
        scratch_types=[
            pltpu.VMEM((BPW,), jnp.int32),
            pltpu.VMEM((2, RING * PACK, S), jnp.float32),
            pltpu.VMEM((BPW, S), jnp.float32),
            pltpu.SemaphoreType.DMA,
            pltpu.SemaphoreType.DMA,
        ],
        compiler_params=pltpu.CompilerParams(
            use_tc_tiling_on_sc=True, needs_layout_passes=False),
    )
    def season_kernel(tab_hbm, idx_hbm, out_hbm, idx_v, ring, srows,
                      sem0, sem1):
        wid = lax.axis_index("s") * NC + lax.axis_index("c")
        base = wid * BPW
        pltpu.sync_copy(idx_hbm.at[pl.ds(base, BPW)], idx_v)
        lanes = lax.iota(jnp.int32, 16)
        n_rounds = BPW // RING
        sems = [sem0, sem1]

        def fire(rnd):
            j0 = rnd * RING
            buf = rnd % 2
            copies = []
            for g16 in range(RING // 16):
                vec = idx_v[pl.ds(j0 + g16 * 16, 16)]
                for l in range(16):
                    j = g16 * 16 + l
                    i = vec[l]
                    copies.append(pltpu.async_copy(
                        tab_hbm.at[i >> 3],
                        ring.at[buf, pl.ds(j * PACK, PACK), :], sems[buf]))
            return copies

        def extract(rnd, copies):
            for c in copies:
                c.wait()
            j0 = rnd * RING
            buf = rnd % 2
            for g in range(RING // 16):
                sl = pl.ds(j0 + g * 16, 16)
                sub = lax.rem(idx_v[sl], PACK)
                rowv = (g * 16 + lanes) * PACK + sub
                bufv = jnp.full((16,), buf, jnp.int32)
                posv = j0 + g * 16 + lanes
                for c in range(S):
                    cv = jnp.full((16,), c, jnp.int32)
                    vals = plsc.load_gather(ring, [bufv, rowv, cv])
                    plsc.store_scatter(srows, [posv, cv], vals)

        pending = fire(0)
        for rnd in range(n_rounds):
            nxt = fire(rnd + 1) if rnd + 1 < n_rounds else None
            extract(rnd, pending)
            pending = nxt
        pltpu.sync_copy(srows, out_hbm.at[pl.ds(base, BPW), :])

    return season_kernel


@functools.lru_cache(maxsize=None)
def _build_small():
    mesh = plsc.VectorSubcoreMesh(core_axis_name="c", subcore_axis_name="s")

    @functools.partial(
        pl.kernel,
        mesh=mesh,
        out_type=(
            jax.ShapeDtypeStruct((B,), jnp.float32),
            jax.ShapeDtypeStruct((B,), jnp.float32),
        ),
        scratch_types=[
            pltpu.VMEM((BPW,), jnp.int32),
            pltpu.VMEM((BPW,), jnp.int32),
            pltpu.VMEM((BPW, PACK), jnp.float32),
            pltpu.VMEM((BPW, PACK), jnp.float32),
            pltpu.VMEM((BPW,), jnp.float32),
            pltpu.VMEM((BPW,), jnp.float32),
            pltpu.SemaphoreType.DMA,
            pltpu.SemaphoreType.DMA,
        ],
        compiler_params=pltpu.CompilerParams(
            use_tc_tiling_on_sc=False, needs_layout_passes=False),
    )
    def small_kernel(lev_hbm, seas_hbm, idx_hbm, lev_out, seas_out,
                     idx_v, row_v, lev_rows, seas_rows, lev_v, seas_v,
                     sem0, sem1):
        wid = lax.axis_index("s") * NC + lax.axis_index("c")
        base = wid * BPW
        pltpu.sync_copy(idx_hbm.at[pl.ds(base, BPW)], idx_v)
        for j in range(BPW // 16):
            sl = pl.ds(j * 16, 16)
            row_v[sl] = lax.shift_right_logical(idx_v[sl], 3)
        c0 = pltpu.async_copy(lev_hbm.at[row_v], lev_rows, sem0)
        c1 = pltpu.async_copy(seas_hbm.at[row_v], seas_rows, sem1)
        c0.wait()
        c1.wait()
        pos0 = lax.iota(jnp.int32, 16)
        for j in range(BPW // 16):
            sl = pl.ds(j * 16, 16)
            lane = lax.rem(idx_v[sl], PACK)
            pos = pos0 + j * 16
            lev_v[sl] = plsc.load_gather(lev_rows, [pos, lane])
            seas_v[sl] = plsc.load_gather(seas_rows, [pos, lane])
        pltpu.sync_copy(lev_v, lev_out.at[pl.ds(base, BPW)])
        pltpu.sync_copy(seas_v, seas_out.at[pl.ds(base, BPW)])

    return small_kernel


def kernel(train, val, test, info_cat, idxs, add_nl_layer,
           init_lev_sms, init_seas_sms, init_seasonalities):
    idx32 = idxs.astype(jnp.int32)
    lev_flat, seas_flat = _build_small()(
        init_lev_sms.reshape(N // PACK, PACK),
        init_seas_sms.reshape(N // PACK, PACK), idx32)
    season = _build_season()(
        init_seasonalities.reshape(N // PACK, PACK, S), idx32)
    return (lev_flat.reshape(B, 1), seas_flat.reshape(B, 1), season)
